# Initial kernel scaffold; baseline (speedup 1.0000x reference)
import jax
import jax.numpy as jnp
from jax.experimental import pallas as pl


def kernel(x, batch, params):
    raise NotImplementedError("write your pallas kernel here")


# speedup vs baseline: 2.1803x; 2.1803x over previous
import jax
import jax.numpy as jnp
from jax.experimental import pallas as pl

_N = 50000
_H = 512
_NSEG = 16
_R = 1000
_NB = _N // _R
_EPS = 1e-5


def _stats_update(st_ref, z):
    @pl.when(pl.program_id(0) == 0)
    def _():
        st_ref[...] = jnp.zeros_like(st_ref)

    st_ref[0:1, :] += jnp.sum(z, axis=0, keepdims=True)
    st_ref[1:2, :] += jnp.sum(z * z, axis=0, keepdims=True)


def _affine_from_stats(st_ref, g_ref, b_ref):
    m = st_ref[0:1, :] * (1.0 / _N)
    v = st_ref[1:2, :] * (1.0 / _N) - m * m
    a = g_ref[...] * jax.lax.rsqrt(v + _EPS)
    c = b_ref[...] - m * a
    return a, c


def _p0_body(x_ref, wf_ref, bf_ref, w1_ref, b1_ref, h_ref, z_ref, st_ref):
    h = jnp.dot(x_ref[...], wf_ref[...], preferred_element_type=jnp.float32) + bf_ref[...]
    z = jnp.dot(h, w1_ref[...], preferred_element_type=jnp.float32) + b1_ref[...]
    h_ref[...] = h
    z_ref[...] = z
    _stats_update(st_ref, z)


def _mid_body(h_ref, z_ref, st_in_ref, g_ref, be_ref, w2_ref, b2_ref, w1_ref, b1_ref,
              ho_ref, zo_ref, st_ref):
    a, c = _affine_from_stats(st_in_ref, g_ref, be_ref)
    zn = jnp.maximum(z_ref[...] * a + c, 0.0)
    hn = h_ref[...] + jnp.dot(zn, w2_ref[...], preferred_element_type=jnp.float32) + b2_ref[...]
    zo = jnp.dot(hn, w1_ref[...], preferred_element_type=jnp.float32) + b1_ref[...]
    ho_ref[...] = hn
    zo_ref[...] = zo
    _stats_update(st_ref, zo)


def _last_body(h_ref, z_ref, st_in_ref, g_ref, be_ref, w2_ref, b2_ref, wp_ref, bp_ref,
               batch_ref, gmax_ref):
    a, c = _affine_from_stats(st_in_ref, g_ref, be_ref)
    zn = jnp.maximum(z_ref[...] * a + c, 0.0)
    hn = h_ref[...] + jnp.dot(zn, w2_ref[...], preferred_element_type=jnp.float32) + b2_ref[...]
    p = jnp.dot(hn, wp_ref[...], preferred_element_type=jnp.float32) + bp_ref[...]

    @pl.when(pl.program_id(0) == 0)
    def _():
        gmax_ref[...] = jnp.full_like(gmax_ref, -jnp.inf)

    ids = batch_ref[...]
    for s in range(_NSEG):
        m = jnp.max(jnp.where(ids == s, p, -jnp.inf), axis=0, keepdims=True)
        gmax_ref[s:s + 1, :] = jnp.maximum(gmax_ref[s:s + 1, :], m)


def _head_body(g0_ref, w1_ref, b1_ref, g1_ref, be1_ref, w2_ref, b2_ref, g2_ref, be2_ref,
               w3_ref, b3_ref, out_ref):
    def bn_relu(t, ga, be):
        m = jnp.mean(t, axis=0, keepdims=True)
        v = jnp.mean((t - m) * (t - m), axis=0, keepdims=True)
        return jnp.maximum(ga * (t - m) * jax.lax.rsqrt(v + _EPS) + be, 0.0)

    t = jnp.dot(g0_ref[...], w1_ref[...], preferred_element_type=jnp.float32) + b1_ref[...]
    t = bn_relu(t, g1_ref[...], be1_ref[...])
    t = jnp.dot(t, w2_ref[...], preferred_element_type=jnp.float32) + b2_ref[...]
    t = bn_relu(t, g2_ref[...], be2_ref[...])
    out_ref[...] = jnp.dot(t, w3_ref[...], preferred_element_type=jnp.float32) + b3_ref[...]


def _row(shape):
    return pl.BlockSpec(shape, lambda i: (i, 0))


def _fix(shape):
    return pl.BlockSpec(shape, lambda i: (0, 0))


_F32 = jnp.float32


def kernel(x, batch, params):
    xp = jnp.pad(x, ((0, 0), (0, 5)))
    wf, bf = params['ffm']
    wfp = jnp.pad(wf, ((0, 5), (0, 0)))
    lps = params['layers']
    w1 = [lp['W1'] for lp in lps]
    b1 = [lp['b1'].reshape(1, _H) for lp in lps]
    ga = [lp['gamma'].reshape(1, _H) for lp in lps]
    be = [lp['beta'].reshape(1, _H) for lp in lps]
    w2 = [lp['alpha'] * lp['W2'] for lp in lps]
    b2 = [(lp['alpha'] * lp['b2']).reshape(1, _H) for lp in lps]
    wp, bp = params['pool']
    bp = bp.reshape(1, _H)
    batch2 = batch.reshape(_N, 1)

    hz = jax.ShapeDtypeStruct((_N, _H), _F32)
    stt = jax.ShapeDtypeStruct((8, _H), _F32)
    rowhz = _row((_R, _H))
    wspec = _fix((_H, _H))
    bspec = _fix((1, _H))
    stspec = _fix((8, _H))

    h, z, st = pl.pallas_call(
        _p0_body,
        grid=(_NB,),
        in_specs=[_row((_R, 8)), _fix((8, _H)), bspec, wspec, bspec],
        out_specs=[rowhz, rowhz, stspec],
        out_shape=[hz, hz, stt],
    )(xp, wfp, bf.reshape(1, _H), w1[0], b1[0])

    for k in (1, 2):
        h, z, st = pl.pallas_call(
            _mid_body,
            grid=(_NB,),
            in_specs=[rowhz, rowhz, stspec, bspec, bspec, wspec, bspec, wspec, bspec],
            out_specs=[rowhz, rowhz, stspec],
            out_shape=[hz, hz, stt],
        )(h, z, st, ga[k - 1], be[k - 1], w2[k - 1], b2[k - 1], w1[k], b1[k])

    gmax = pl.pallas_call(
        _last_body,
        grid=(_NB,),
        in_specs=[rowhz, rowhz, stspec, bspec, bspec, wspec, bspec, wspec, bspec,
                  _row((_R, 1))],
        out_specs=_fix((_NSEG, _H)),
        out_shape=jax.ShapeDtypeStruct((_NSEG, _H), _F32),
    )(h, z, st, ga[2], be[2], w2[2], b2[2], wp, bp, batch2)

    rw1, rb1 = params['reg_W1']
    rw2, rb2 = params['reg_W2']
    rw3, rb3 = params['reg_W3']
    out = pl.pallas_call(
        _head_body,
        grid=(1,),
        in_specs=[_fix((_NSEG, _H)), wspec, bspec, bspec, bspec, wspec, bspec,
                  bspec, bspec, wspec, bspec],
        out_specs=_fix((_NSEG, _H)),
        out_shape=jax.ShapeDtypeStruct((_NSEG, _H), _F32),
    )(gmax, rw1, rb1.reshape(1, _H), params['reg_g1'].reshape(1, _H),
      params['reg_b1'].reshape(1, _H), rw2, rb2.reshape(1, _H),
      params['reg_g2'].reshape(1, _H), params['reg_b2'].reshape(1, _H),
      rw3, rb3.reshape(1, _H))
    return out


# speedup vs baseline: 2.8925x; 1.3266x over previous
import jax
import jax.numpy as jnp
from jax.experimental import pallas as pl

_N = 50000
_H = 512
_NSEG = 16
_R = 1000
_NB = _N // _R
_EPS = 1e-5

_F32 = jnp.float32
_BF16 = jnp.bfloat16


def _stats_update(st_ref, z32):
    @pl.when(pl.program_id(0) == 0)
    def _():
        st_ref[...] = jnp.zeros_like(st_ref)

    st_ref[0:1, :] += jnp.sum(z32, axis=0, keepdims=True)
    st_ref[1:2, :] += jnp.sum(z32 * z32, axis=0, keepdims=True)


def _affine_from_stats(st_ref, g_ref, b_ref):
    m = st_ref[0:1, :] * (1.0 / _N)
    v = st_ref[1:2, :] * (1.0 / _N) - m * m
    a = g_ref[...] * jax.lax.rsqrt(v + _EPS)
    c = b_ref[...] - m * a
    return a, c


def _bdot(a, b):
    return jnp.dot(a, b, preferred_element_type=_F32)


def _p0_body(x_ref, wf_ref, bf_ref, w1_ref, b1_ref, h_ref, z_ref, st_ref):
    h = _bdot(x_ref[...], wf_ref[...]) + bf_ref[...]
    z = _bdot(h.astype(_BF16), w1_ref[...]) + b1_ref[...]
    h_ref[...] = h
    z_ref[...] = z.astype(_BF16)
    _stats_update(st_ref, z)


def _mid_body(h_ref, z_ref, st_in_ref, g_ref, be_ref, w2_ref, b2_ref, w1_ref, b1_ref,
              ho_ref, zo_ref, st_ref):
    a, c = _affine_from_stats(st_in_ref, g_ref, be_ref)
    zn = jnp.maximum(z_ref[...].astype(_F32) * a + c, 0.0)
    hn = h_ref[...] + _bdot(zn.astype(_BF16), w2_ref[...]) + b2_ref[...]
    zo = _bdot(hn.astype(_BF16), w1_ref[...]) + b1_ref[...]
    ho_ref[...] = hn
    zo_ref[...] = zo.astype(_BF16)
    _stats_update(st_ref, zo)


def _last_body(h_ref, z_ref, st_in_ref, g_ref, be_ref, w2_ref, b2_ref, wp_ref, bp_ref,
               batch_ref, gmax_ref):
    a, c = _affine_from_stats(st_in_ref, g_ref, be_ref)
    zn = jnp.maximum(z_ref[...].astype(_F32) * a + c, 0.0)
    hn = h_ref[...] + _bdot(zn.astype(_BF16), w2_ref[...]) + b2_ref[...]
    p = _bdot(hn, wp_ref[...]) + bp_ref[...]

    @pl.when(pl.program_id(0) == 0)
    def _():
        gmax_ref[...] = jnp.full_like(gmax_ref, -jnp.inf)

    ids = batch_ref[...]
    lo = batch_ref[0, 0]
    hi = batch_ref[_R - 1, 0]
    for s in range(_NSEG):
        @pl.when((lo <= s) & (s <= hi))
        def _(s=s):
            m = jnp.max(jnp.where(ids == s, p, -jnp.inf), axis=0, keepdims=True)
            gmax_ref[s:s + 1, :] = jnp.maximum(gmax_ref[s:s + 1, :], m)


def _head_body(g0_ref, w1_ref, b1_ref, g1_ref, be1_ref, w2_ref, b2_ref, g2_ref, be2_ref,
               w3_ref, b3_ref, out_ref):
    def bn_relu(t, ga, be):
        m = jnp.mean(t, axis=0, keepdims=True)
        v = jnp.mean((t - m) * (t - m), axis=0, keepdims=True)
        return jnp.maximum(ga * (t - m) * jax.lax.rsqrt(v + _EPS) + be, 0.0)

    t = _bdot(g0_ref[...], w1_ref[...]) + b1_ref[...]
    t = bn_relu(t, g1_ref[...], be1_ref[...])
    t = _bdot(t, w2_ref[...]) + b2_ref[...]
    t = bn_relu(t, g2_ref[...], be2_ref[...])
    out_ref[...] = _bdot(t, w3_ref[...]) + b3_ref[...]


def _row(shape):
    return pl.BlockSpec(shape, lambda i: (i, 0))


def _fix(shape):
    return pl.BlockSpec(shape, lambda i: (0, 0))


def kernel(x, batch, params):
    xp = jnp.pad(x, ((0, 0), (0, 5)))
    wf, bf = params['ffm']
    wfp = jnp.pad(wf, ((0, 5), (0, 0)))
    lps = params['layers']
    w1 = [lp['W1'].astype(_BF16) for lp in lps]
    b1 = [lp['b1'].reshape(1, _H) for lp in lps]
    ga = [lp['gamma'].reshape(1, _H) for lp in lps]
    be = [lp['beta'].reshape(1, _H) for lp in lps]
    w2 = [(lp['alpha'] * lp['W2']).astype(_BF16) for lp in lps]
    b2 = [(lp['alpha'] * lp['b2']).reshape(1, _H) for lp in lps]
    wp, bp = params['pool']
    bp = bp.reshape(1, _H)
    batch2 = batch.reshape(_N, 1)

    hf = jax.ShapeDtypeStruct((_N, _H), _F32)
    zb = jax.ShapeDtypeStruct((_N, _H), _BF16)
    stt = jax.ShapeDtypeStruct((8, _H), _F32)
    rowhz = _row((_R, _H))
    wspec = _fix((_H, _H))
    bspec = _fix((1, _H))
    stspec = _fix((8, _H))

    h, z, st = pl.pallas_call(
        _p0_body,
        grid=(_NB,),
        in_specs=[_row((_R, 8)), _fix((8, _H)), bspec, wspec, bspec],
        out_specs=[rowhz, rowhz, stspec],
        out_shape=[hf, zb, stt],
    )(xp, wfp, bf.reshape(1, _H), w1[0], b1[0])

    for k in (1, 2):
        h, z, st = pl.pallas_call(
            _mid_body,
            grid=(_NB,),
            in_specs=[rowhz, rowhz, stspec, bspec, bspec, wspec, bspec, wspec, bspec],
            out_specs=[rowhz, rowhz, stspec],
            out_shape=[hf, zb, stt],
        )(h, z, st, ga[k - 1], be[k - 1], w2[k - 1], b2[k - 1], w1[k], b1[k])

    gmax = pl.pallas_call(
        _last_body,
        grid=(_NB,),
        in_specs=[rowhz, rowhz, stspec, bspec, bspec, wspec, bspec, wspec, bspec,
                  _row((_R, 1))],
        out_specs=_fix((_NSEG, _H)),
        out_shape=jax.ShapeDtypeStruct((_NSEG, _H), _F32),
    )(h, z, st, ga[2], be[2], w2[2], b2[2], wp, bp, batch2)

    rw1, rb1 = params['reg_W1']
    rw2, rb2 = params['reg_W2']
    rw3, rb3 = params['reg_W3']
    out = pl.pallas_call(
        _head_body,
        grid=(1,),
        in_specs=[_fix((_NSEG, _H)), wspec, bspec, bspec, bspec, wspec, bspec,
                  bspec, bspec, wspec, bspec],
        out_specs=_fix((_NSEG, _H)),
        out_shape=jax.ShapeDtypeStruct((_NSEG, _H), _F32),
    )(gmax, rw1, rb1.reshape(1, _H), params['reg_g1'].reshape(1, _H),
      params['reg_b1'].reshape(1, _H), rw2, rb2.reshape(1, _H),
      params['reg_g2'].reshape(1, _H), params['reg_b2'].reshape(1, _H),
      rw3, rb3.reshape(1, _H))
    return out


# speedup vs baseline: 3.4644x; 1.1977x over previous
import jax
import jax.numpy as jnp
from jax.experimental import pallas as pl
from jax.experimental.pallas import tpu as pltpu

_N = 50000
_H = 512
_NSEG = 16
_R = 1000
_NB = _N // _R
_EPS = 1e-5

_F32 = jnp.float32
_BF16 = jnp.bfloat16


def _bdot(a, b):
    return jnp.dot(a, b, preferred_element_type=_F32)


def _hdot(a, b):
    return jnp.dot(a, b, preferred_element_type=_F32,
                   precision=jax.lax.Precision.HIGHEST)


def _colsum(v):
    return jnp.sum(v, axis=0, keepdims=True)


def _aprime(wfp_ref, w_ref, bias_ref):
    a = _hdot(wfp_ref[...], w_ref[...])
    row = jax.lax.broadcasted_iota(jnp.int32, (8, _H), 0)
    return a + jnp.where(row == 3, bias_ref[...], 0.0)


def _zstats(p_ref, ap, w1f, w1b, sd, xd_gd_small):
    sum_z = _hdot(p_ref[3:4, :], ap) + sd
    quad_x = _colsum(ap * _hdot(p_ref[0:8, :], ap))
    return sum_z, quad_x + xd_gd_small


def _affine(sum_z, sumsq_z, ga_ref, be_ref, ac_ref):
    mean = sum_z * (1.0 / _N)
    v = sumsq_z * (1.0 / _N) - mean * mean
    a = ga_ref[...] * jax.lax.rsqrt(v + _EPS)
    c = be_ref[...] - mean * a
    ac_ref[0:1, :] = a
    ac_ref[1:2, :] = c


def _pp_body(x_ref, p_ref):
    @pl.when(pl.program_id(0) == 0)
    def _():
        p_ref[...] = jnp.zeros_like(p_ref)

    xb = x_ref[...]
    p_ref[...] += jax.lax.dot_general(
        xb, xb, (((0,), (0,)), ((), ())), preferred_element_type=_F32)


def _acc_d(x_ref, dn, sx_ref):
    @pl.when(pl.program_id(0) == 0)
    def _():
        sx_ref[...] = jnp.zeros_like(sx_ref)

    sx_ref[0:8, :] += jax.lax.dot_general(
        x_ref[...], dn, (((0,), (0,)), ((), ())), preferred_element_type=_F32)
    sx_ref[8:9, :] += _colsum(dn)


def _l1_body(x_ref, p_ref, wfp_ref, w1f_ref, b1_ref, ga_ref, be_ref,
             w2_ref, b2_ref, d_ref, sx_ref, ac_ref, ap_ref):
    @pl.when(pl.program_id(0) == 0)
    def _():
        ap_ref[...] = _aprime(wfp_ref, w1f_ref, b1_ref)
        sum_z, sumsq_z = _zstats(p_ref, ap_ref[...], None, None, 0.0, 0.0)
        _affine(sum_z, sumsq_z, ga_ref, be_ref, ac_ref)

    z = _bdot(x_ref[...], ap_ref[...])
    zn = jnp.maximum(z * ac_ref[0:1, :] + ac_ref[1:2, :], 0.0)
    dn = _bdot(zn.astype(_BF16), w2_ref[...]) + b2_ref[...]
    d_ref[...] = dn.astype(_BF16)
    _acc_d(x_ref, dn, sx_ref)


def _l2_body(x_ref, p_ref, d_ref, sx_ref, wfp_ref,
             w1f_ref, w1b_ref, b1_ref, ga_ref, be_ref, w2_ref, b2_ref,
             do_ref, sxo_ref, ac_ref, ap_ref):
    @pl.when(pl.program_id(0) == 0)
    def _():
        ap = _aprime(wfp_ref, w1f_ref, b1_ref)
        ap_ref[...] = ap
        w1b = w1b_ref[...]
        sd = _hdot(sx_ref[8:9, :], w1b.astype(_F32))
        cross = 2.0 * _colsum(ap * _hdot(sx_ref[0:8, :], w1f_ref[...]))
        sum_z, sumsq_z = _zstats(p_ref, ap, None, None, sd, cross)
        _affine(sum_z, sumsq_z, ga_ref, be_ref, ac_ref)

    d32 = d_ref[...].astype(_F32)
    z = _bdot(x_ref[...], ap_ref[...]) + _bdot(d_ref[...], w1b_ref[...])
    zn = jnp.maximum(z * ac_ref[0:1, :] + ac_ref[1:2, :], 0.0)
    dn = d32 + _bdot(zn.astype(_BF16), w2_ref[...]) + b2_ref[...]
    dnb = dn.astype(_BF16)
    do_ref[...] = dnb
    _acc_d(x_ref, dn, sxo_ref)


def _l3_body(x_ref, p_ref, d_ref, sx_ref, wfp_ref,
             w1f_ref, w1b_ref, b1_ref, ga_ref, be_ref, w2_ref, b2_ref,
             wpf_ref, wpb_ref, bp_ref, batch_ref,
             rw1_ref, rb1_ref, rg1_ref, rbe1_ref, rw2_ref, rb2_ref, rg2_ref,
             rbe2_ref, rw3_ref, rb3_ref,
             gmax_ref, out_ref, ac_ref, ap_ref, app_ref):
    @pl.when(pl.program_id(0) == 0)
    def _():
        ap = _aprime(wfp_ref, w1f_ref, b1_ref)
        ap_ref[...] = ap
        app_ref[...] = _aprime(wfp_ref, wpf_ref, bp_ref)
        w1b = w1b_ref[...]
        sd = _hdot(sx_ref[8:9, :], w1b.astype(_F32))
        cross = 2.0 * _colsum(ap * _hdot(sx_ref[0:8, :], w1f_ref[...]))
        sum_z, sumsq_z = _zstats(p_ref, ap, None, None, sd, cross)
        _affine(sum_z, sumsq_z, ga_ref, be_ref, ac_ref)

    d32 = d_ref[...].astype(_F32)
    z = _bdot(x_ref[...], ap_ref[...]) + _bdot(d_ref[...], w1b_ref[...])
    zn = jnp.maximum(z * ac_ref[0:1, :] + ac_ref[1:2, :], 0.0)
    dn = d32 + _bdot(zn.astype(_BF16), w2_ref[...]) + b2_ref[...]
    xv = x_ref[...]
    p = _bdot(dn.astype(_BF16), wpb_ref[...])
    for k in range(8):
        p += xv[:, k:k + 1] * app_ref[k:k + 1, :]

    @pl.when(pl.program_id(0) == 0)
    def _():
        gmax_ref[...] = jnp.full_like(gmax_ref, -jnp.inf)

    ids = batch_ref[...]
    lo = batch_ref[0, 0]
    hi = batch_ref[_R - 1, 0]

    @pl.when(lo == hi)
    def _():
        m = jnp.max(p, axis=0, keepdims=True)
        rows = jax.lax.broadcasted_iota(jnp.int32, (_NSEG, _H), 0)
        upd = jnp.where(rows == lo, m, -jnp.inf)
        gmax_ref[...] = jnp.maximum(gmax_ref[...], upd)

    @pl.when(lo != hi)
    def _():
        for seg in range(_NSEG):
            @pl.when((lo <= seg) & (seg <= hi))
            def _(seg=seg):
                m = jnp.max(jnp.where(ids == seg, p, -jnp.inf), axis=0, keepdims=True)
                gmax_ref[seg:seg + 1, :] = jnp.maximum(gmax_ref[seg:seg + 1, :], m)

    @pl.when(pl.program_id(0) == _NB - 1)
    def _():
        def bn_relu(t, gaa, bee):
            m = jnp.mean(t, axis=0, keepdims=True)
            v = jnp.mean((t - m) * (t - m), axis=0, keepdims=True)
            return jnp.maximum(gaa * (t - m) * jax.lax.rsqrt(v + _EPS) + bee, 0.0)

        t = _hdot(gmax_ref[...], rw1_ref[...]) + rb1_ref[...]
        t = bn_relu(t, rg1_ref[...], rbe1_ref[...])
        t = _hdot(t, rw2_ref[...]) + rb2_ref[...]
        t = bn_relu(t, rg2_ref[...], rbe2_ref[...])
        out_ref[...] = _hdot(t, rw3_ref[...]) + rb3_ref[...]


def _row(shape):
    return pl.BlockSpec(shape, lambda i: (i, 0))


def _fix(shape):
    return pl.BlockSpec(shape, lambda i: (0, 0))


def kernel(x, batch, params):
    one = jnp.ones((_N, 1), _F32)
    zero4 = jnp.zeros((_N, 4), _F32)
    xp = jnp.concatenate([x, one, zero4], axis=1)
    wf, bf = params['ffm']
    wfp = jnp.concatenate([wf, bf.reshape(1, _H), jnp.zeros((4, _H), _F32)], axis=0)
    lps = params['layers']
    w1f = [lp['W1'] for lp in lps]
    w1b = [lp['W1'].astype(_BF16) for lp in lps]
    b1 = [lp['b1'].reshape(1, _H) for lp in lps]
    ga = [lp['gamma'].reshape(1, _H) for lp in lps]
    be = [lp['beta'].reshape(1, _H) for lp in lps]
    w2 = [(lp['alpha'] * lp['W2']).astype(_BF16) for lp in lps]
    b2 = [(lp['alpha'] * lp['b2']).reshape(1, _H) for lp in lps]
    wpf, bp = params['pool']
    wpb = wpf.astype(_BF16)
    bp = bp.reshape(1, _H)
    batch2 = batch.reshape(_N, 1)

    dsh = jax.ShapeDtypeStruct((_N, _H), _BF16)
    sxsh = jax.ShapeDtypeStruct((16, _H), _F32)
    rowd = _row((_R, _H))
    rowx = _row((_R, 8))
    wspec = _fix((_H, _H))
    w8 = _fix((8, _H))
    bspec = _fix((1, _H))
    sxspec = _fix((16, _H))
    pspec = _fix((8, 8))
    ac_scr = pltpu.VMEM((8, _H), _F32)
    ap_scr = pltpu.VMEM((8, _H), _F32)

    p = pl.pallas_call(
        _pp_body,
        grid=(5,),
        in_specs=[pl.BlockSpec((_N // 5, 8), lambda i: (i, 0))],
        out_specs=pspec,
        out_shape=jax.ShapeDtypeStruct((8, 8), _F32),
    )(xp)

    d, sx = pl.pallas_call(
        _l1_body,
        grid=(_NB,),
        in_specs=[rowx, pspec, w8, wspec, bspec, bspec, bspec, wspec, bspec],
        out_specs=[rowd, sxspec],
        out_shape=[dsh, sxsh],
        scratch_shapes=[ac_scr, ap_scr],
    )(xp, p, wfp, w1f[0], b1[0], ga[0], be[0], w2[0], b2[0])

    d, sx = pl.pallas_call(
        _l2_body,
        grid=(_NB,),
        in_specs=[rowx, pspec, rowd, sxspec, w8,
                  wspec, wspec, bspec, bspec, bspec, wspec, bspec],
        out_specs=[rowd, sxspec],
        out_shape=[dsh, sxsh],
        scratch_shapes=[ac_scr, ap_scr],
    )(xp, p, d, sx, wfp, w1f[1], w1b[1], b1[1], ga[1], be[1], w2[1], b2[1])

    rw1, rb1 = params['reg_W1']
    rw2, rb2 = params['reg_W2']
    rw3, rb3 = params['reg_W3']
    gmax, out = pl.pallas_call(
        _l3_body,
        grid=(_NB,),
        in_specs=[rowx, pspec, rowd, sxspec, w8,
                  wspec, wspec, bspec, bspec, bspec, wspec, bspec,
                  wspec, wspec, bspec, _row((_R, 1)),
                  wspec, bspec, bspec, bspec, wspec, bspec, bspec, bspec,
                  wspec, bspec],
        out_specs=[_fix((_NSEG, _H)), _fix((_NSEG, _H))],
        out_shape=[jax.ShapeDtypeStruct((_NSEG, _H), _F32),
                   jax.ShapeDtypeStruct((_NSEG, _H), _F32)],
        scratch_shapes=[ac_scr, ap_scr, pltpu.VMEM((8, _H), _F32)],
    )(xp, p, d, sx, wfp, w1f[2], w1b[2], b1[2], ga[2], be[2], w2[2], b2[2],
      wpf, wpb, bp, batch2,
      rw1, rb1.reshape(1, _H), params['reg_g1'].reshape(1, _H),
      params['reg_b1'].reshape(1, _H), rw2, rb2.reshape(1, _H),
      params['reg_g2'].reshape(1, _H), params['reg_b2'].reshape(1, _H),
      rw3, rb3.reshape(1, _H))
    return out
